# Initial kernel scaffold; baseline (speedup 1.0000x reference)
import jax
import jax.numpy as jnp
from jax.experimental import pallas as pl


def kernel(x, edge_index, edge_attr, W_msg, b_msg, W_upd, b_upd, W_out, b_out):
    raise NotImplementedError("write your pallas kernel here")


# speedup vs baseline: 1.4191x; 1.4191x over previous
import functools

import jax
import jax.numpy as jnp
from jax import lax
from jax.experimental import pallas as pl
from jax.experimental.pallas import tpu as pltpu
from jax.experimental.pallas import tpu_sc as plsc


def _c_body(ea_ref, w_ref, b_ref, out_ref):
    out_ref[...] = (
        jnp.dot(ea_ref[...], w_ref[...], preferred_element_type=jnp.float32)
        + b_ref[...]
    )


def _edge_bias_table(edge_attr, W3, b_msg, E, ED, H):
    EB = 8000
    return pl.pallas_call(
        _c_body,
        grid=(E // EB,),
        in_specs=[
            pl.BlockSpec((EB, ED), lambda i: (i, 0)),
            pl.BlockSpec((ED, H), lambda i: (0, 0)),
            pl.BlockSpec((1, H), lambda i: (0, 0)),
        ],
        out_specs=pl.BlockSpec((EB, H), lambda i: (i, 0)),
        out_shape=jax.ShapeDtypeStruct((E, H), jnp.float32),
    )(edge_attr, W3, b_msg.reshape(1, H))


def _ab_body(h_ref, w1_ref, w2_ref, a_ref, b_ref):
    h = h_ref[...]
    a_ref[...] = jnp.dot(h, w1_ref[...], preferred_element_type=jnp.float32)
    b_ref[...] = jnp.dot(h, w2_ref[...], preferred_element_type=jnp.float32)


def _proj_ab(h, W1, W2, N, H):
    NB = 2000
    nspec = pl.BlockSpec((NB, H), lambda i: (i, 0))
    nshape = jax.ShapeDtypeStruct((N, H), jnp.float32)
    return pl.pallas_call(
        _ab_body,
        grid=(N // NB,),
        in_specs=[
            nspec,
            pl.BlockSpec((H, H), lambda i: (0, 0)),
            pl.BlockSpec((H, H), lambda i: (0, 0)),
        ],
        out_specs=[nspec, nspec],
        out_shape=[nshape, nshape],
    )(h, W1, W2)


def _upd_mid_body(p_ref, mp_ref, h_ref, wum_ref, wuh_ref, bu_ref, w1_ref,
                  w2_ref, m_ref, hn_ref, a_ref, b_ref):
    m = mp_ref[...] + p_ref[...]
    hn = jnp.tanh(
        jnp.dot(m, wum_ref[...], preferred_element_type=jnp.float32)
        + jnp.dot(h_ref[...], wuh_ref[...], preferred_element_type=jnp.float32)
        + bu_ref[...]
    )
    m_ref[...] = m
    hn_ref[...] = hn
    a_ref[...] = jnp.dot(hn, w1_ref[...], preferred_element_type=jnp.float32)
    b_ref[...] = jnp.dot(hn, w2_ref[...], preferred_element_type=jnp.float32)


def _update_mid(P, m_prev, h, Wum, Wuh, b_upd, W1, W2, N, H):
    NB = 2000
    wspec = pl.BlockSpec((H, H), lambda i: (0, 0))
    nspec = pl.BlockSpec((NB, H), lambda i: (i, 0))
    nshape = jax.ShapeDtypeStruct((N, H), jnp.float32)
    return pl.pallas_call(
        _upd_mid_body,
        grid=(N // NB,),
        in_specs=[
            nspec, nspec, nspec, wspec, wspec,
            pl.BlockSpec((1, H), lambda i: (0, 0)),
            wspec, wspec,
        ],
        out_specs=[nspec, nspec, nspec, nspec],
        out_shape=[nshape, nshape, nshape, nshape],
    )(P, m_prev, h, Wum, Wuh, b_upd.reshape(1, H), W1, W2)


def _readout_body(h_ref, wo_ref, bo_ref, out_ref, acc_ref):
    i = pl.program_id(0)
    part = jnp.sum(h_ref[...], axis=0, keepdims=True)

    @pl.when(i == 0)
    def _():
        acc_ref[...] = part

    @pl.when(i > 0)
    def _():
        acc_ref[...] = acc_ref[...] + part

    @pl.when(i == pl.num_programs(0) - 1)
    def _():
        out_ref[...] = jnp.tanh(
            jnp.dot(acc_ref[...], wo_ref[...], preferred_element_type=jnp.float32)
            + bo_ref[...]
        )


def _readout(h, W_out, b_out, N, H):
    NB = 2000
    return pl.pallas_call(
        _readout_body,
        grid=(N // NB,),
        in_specs=[
            pl.BlockSpec((NB, H), lambda i: (i, 0)),
            pl.BlockSpec((H, H), lambda i: (0, 0)),
            pl.BlockSpec((1, H), lambda i: (0, 0)),
        ],
        out_specs=pl.BlockSpec((1, H), lambda i: (0, 0)),
        out_shape=jax.ShapeDtypeStruct((1, H), jnp.float32),
        scratch_shapes=[pltpu.VMEM((1, H), jnp.float32)],
    )(h, W_out, b_out.reshape(1, H))


def _make_edge_kernel(N, E, H):
    info = plsc.get_sparse_core_info()
    NC, NS = info.num_cores, info.num_subcores
    N2 = N // 2
    EPT = E // NS
    CHUNK = 32
    NCHUNKS = EPT // CHUNK
    ZROWS = 200
    AROWS = N2 + ZROWS
    NZCHUNKS = AROWS // ZROWS
    NOCHUNKS = N2 // ZROWS
    assert NC == 2 and EPT * NS == E and CHUNK * NCHUNKS == EPT
    assert CHUNK % 16 == 0 and ZROWS % 8 == 0
    assert NZCHUNKS * ZROWS == AROWS and NOCHUNKS * ZROWS == N2

    mesh = plsc.VectorSubcoreMesh(core_axis_name="c", subcore_axis_name="s")

    @functools.partial(
        pl.kernel,
        out_type=jax.ShapeDtypeStruct((N, H), jnp.float32),
        mesh=mesh,
        scratch_types=[
            pltpu.VMEM((CHUNK,), jnp.int32),
            pltpu.VMEM((CHUNK,), jnp.int32),
            pltpu.VMEM((CHUNK,), jnp.int32),
            pltpu.VMEM((CHUNK, H), jnp.float32),
            pltpu.VMEM((CHUNK, H), jnp.float32),
            pltpu.VMEM((CHUNK, H), jnp.float32),
            pltpu.VMEM((CHUNK, H), jnp.float32),
            pltpu.VMEM((ZROWS, H), jnp.float32),
            pltpu.VMEM_SHARED((AROWS, H), jnp.float32),
            pltpu.SemaphoreType.DMA,
            pltpu.SemaphoreType.DMA,
            pltpu.SemaphoreType.DMA,
        ],
    )
    def edge_kernel(a_hbm, b_hbm, c_hbm, dst_hbm, src_hbm, out_hbm,
                    di_v, dl_v, si_v, ar_v, br_v, cr_v, mg_v, zb_v, m_sh,
                    sema, semb, semc):
        cid = lax.axis_index("c")
        sid = lax.axis_index("s")
        nvec = H // 16

        zval = jnp.zeros((16,), jnp.float32)

        def zb_body(j, _):
            zb_v[j // nvec, pl.ds((j % nvec) * 16, 16)] = zval
            return 0

        lax.fori_loop(0, ZROWS * nvec, zb_body, 0)
        for k in range((NZCHUNKS + NS - 1) // NS):
            zc = sid + k * NS

            @pl.when(zc < NZCHUNKS)
            def _():
                pltpu.sync_copy(
                    zb_v, m_sh.at[pl.ds(pl.multiple_of(zc * ZROWS, 8), ZROWS)])
        plsc.subcore_barrier()

        base_t = sid * EPT
        nbase = cid * N2

        def chunk_body(i, _):
            base = pl.multiple_of(base_t + i * CHUNK, 8)
            pltpu.sync_copy(dst_hbm.at[pl.ds(base, CHUNK)], di_v)
            pltpu.sync_copy(src_hbm.at[pl.ds(base, CHUNK)], si_v)

            def adj_body(k, _):
                sl = pl.ds(k * 16, 16)
                loc = di_v[sl] - nbase
                ok = jnp.logical_and(loc >= 0, loc < N2)
                dl_v[sl] = jnp.where(ok, loc, N2)
                return 0

            lax.fori_loop(0, CHUNK // 16, adj_body, 0)
            cpa = pltpu.async_copy(a_hbm.at[di_v], ar_v, sema)
            cpb = pltpu.async_copy(b_hbm.at[si_v], br_v, semb)
            cpc = pltpu.async_copy(c_hbm.at[pl.ds(base, CHUNK)], cr_v, semc)
            cpa.wait()
            cpb.wait()
            cpc.wait()

            def row_body(r, _):
                for v in range(nvec):
                    sl = pl.ds(v * 16, 16)
                    z = ar_v[r, sl] + br_v[r, sl] + cr_v[r, sl]
                    e = jnp.exp(jnp.abs(z) * -2.0)
                    t = (1.0 - e) / (1.0 + e)
                    mg_v[r, sl] = jnp.where(z < 0.0, -t, t)
                return 0

            lax.fori_loop(0, CHUNK, row_body, 0)
            pltpu.sync_copy(mg_v, m_sh.at[dl_v], add=True)
            return 0

        lax.fori_loop(0, NCHUNKS, chunk_body, 0)
        plsc.subcore_barrier()

        for k in range((NOCHUNKS + NS - 1) // NS):
            zc = sid + k * NS

            @pl.when(zc < NOCHUNKS)
            def _():
                off = pl.multiple_of(zc * ZROWS, 8)
                pltpu.sync_copy(m_sh.at[pl.ds(off, ZROWS)], zb_v)
                pltpu.sync_copy(
                    zb_v,
                    out_hbm.at[pl.ds(pl.multiple_of(nbase + off, 8), ZROWS)])

    return edge_kernel


def kernel(x, edge_index, edge_attr, W_msg, b_msg, W_upd, b_upd, W_out, b_out):
    N, H = x.shape
    E = edge_index.shape[1]
    ED = edge_attr.shape[1]

    W1 = W_msg[:H]
    W2 = W_msg[H:2 * H]
    W3 = W_msg[2 * H:]
    Wum = W_upd[:H]
    Wuh = W_upd[H:]
    dst = edge_index[0]
    src = edge_index[1]

    C = _edge_bias_table(edge_attr, W3, b_msg, E, ED, H)
    A0, B0 = _proj_ab(x, W1, W2, N, H)

    edge_kernel = _make_edge_kernel(N, E, H)

    def body(carry, _):
        m, h, A, B = carry
        P = edge_kernel(A, B, C, dst, src)
        m, h, A, B = _update_mid(P, m, h, Wum, Wuh, b_upd, W1, W2, N, H)
        return (m, h, A, B), None

    m0 = jnp.zeros((N, H), jnp.float32)
    (m, h, A, B), _ = lax.scan(body, (m0, x, A0, B0), None, length=2)
    out = _readout(h, W_out, b_out, N, H)
    return out.reshape(H)


# speedup vs baseline: 3.6927x; 2.6023x over previous
import functools

import jax
import jax.numpy as jnp
from jax import lax
from jax.experimental import pallas as pl
from jax.experimental.pallas import tpu as pltpu
from jax.experimental.pallas import tpu_sc as plsc


def _c_body(ea_ref, w_ref, b_ref, out_ref):
    out_ref[...] = (
        jnp.dot(ea_ref[...], w_ref[...], preferred_element_type=jnp.float32)
        + b_ref[...]
    )


def _edge_bias_table(edge_attr, W3, b_msg, E, ED, H):
    EB = 8000
    return pl.pallas_call(
        _c_body,
        grid=(E // EB,),
        in_specs=[
            pl.BlockSpec((EB, ED), lambda i: (i, 0)),
            pl.BlockSpec((ED, H), lambda i: (0, 0)),
            pl.BlockSpec((1, H), lambda i: (0, 0)),
        ],
        out_specs=pl.BlockSpec((EB, H), lambda i: (i, 0)),
        out_shape=jax.ShapeDtypeStruct((E, H), jnp.float32),
    )(edge_attr, W3, b_msg.reshape(1, H))


def _ab_body(h_ref, w1_ref, w2_ref, a_ref, b_ref):
    h = h_ref[...]
    a_ref[...] = jnp.dot(h, w1_ref[...], preferred_element_type=jnp.float32)
    b_ref[...] = jnp.dot(h, w2_ref[...], preferred_element_type=jnp.float32)


def _proj_ab(h, W1, W2, N, H):
    NB = 2000
    nspec = pl.BlockSpec((NB, H), lambda i: (i, 0))
    nshape = jax.ShapeDtypeStruct((N, H), jnp.float32)
    return pl.pallas_call(
        _ab_body,
        grid=(N // NB,),
        in_specs=[
            nspec,
            pl.BlockSpec((H, H), lambda i: (0, 0)),
            pl.BlockSpec((H, H), lambda i: (0, 0)),
        ],
        out_specs=[nspec, nspec],
        out_shape=[nshape, nshape],
    )(h, W1, W2)


def _upd_mid_body(p_ref, mp_ref, h_ref, wum_ref, wuh_ref, bu_ref, w1_ref,
                  w2_ref, m_ref, hn_ref, a_ref, b_ref):
    m = mp_ref[...] + p_ref[...]
    hn = jnp.tanh(
        jnp.dot(m, wum_ref[...], preferred_element_type=jnp.float32)
        + jnp.dot(h_ref[...], wuh_ref[...], preferred_element_type=jnp.float32)
        + bu_ref[...]
    )
    m_ref[...] = m
    hn_ref[...] = hn
    a_ref[...] = jnp.dot(hn, w1_ref[...], preferred_element_type=jnp.float32)
    b_ref[...] = jnp.dot(hn, w2_ref[...], preferred_element_type=jnp.float32)


def _update_mid(P, m_prev, h, Wum, Wuh, b_upd, W1, W2, N, H):
    NB = 2000
    wspec = pl.BlockSpec((H, H), lambda i: (0, 0))
    nspec = pl.BlockSpec((NB, H), lambda i: (i, 0))
    nshape = jax.ShapeDtypeStruct((N, H), jnp.float32)
    return pl.pallas_call(
        _upd_mid_body,
        grid=(N // NB,),
        in_specs=[
            nspec, nspec, nspec, wspec, wspec,
            pl.BlockSpec((1, H), lambda i: (0, 0)),
            wspec, wspec,
        ],
        out_specs=[nspec, nspec, nspec, nspec],
        out_shape=[nshape, nshape, nshape, nshape],
    )(P, m_prev, h, Wum, Wuh, b_upd.reshape(1, H), W1, W2)


def _readout_body(h_ref, wo_ref, bo_ref, out_ref, acc_ref):
    i = pl.program_id(0)
    part = jnp.sum(h_ref[...], axis=0, keepdims=True)

    @pl.when(i == 0)
    def _():
        acc_ref[...] = part

    @pl.when(i > 0)
    def _():
        acc_ref[...] = acc_ref[...] + part

    @pl.when(i == pl.num_programs(0) - 1)
    def _():
        out_ref[...] = jnp.tanh(
            jnp.dot(acc_ref[...], wo_ref[...], preferred_element_type=jnp.float32)
            + bo_ref[...]
        )


def _readout(h, W_out, b_out, N, H):
    NB = 2000
    return pl.pallas_call(
        _readout_body,
        grid=(N // NB,),
        in_specs=[
            pl.BlockSpec((NB, H), lambda i: (i, 0)),
            pl.BlockSpec((H, H), lambda i: (0, 0)),
            pl.BlockSpec((1, H), lambda i: (0, 0)),
        ],
        out_specs=pl.BlockSpec((1, H), lambda i: (0, 0)),
        out_shape=jax.ShapeDtypeStruct((1, H), jnp.float32),
        scratch_shapes=[pltpu.VMEM((1, H), jnp.float32)],
    )(h, W_out, b_out.reshape(1, H))


def _make_edge_kernel(N, E, H):
    info = plsc.get_sparse_core_info()
    NC, NS = info.num_cores, info.num_subcores
    N2 = N // 2
    EPT = E // NS
    CHUNK = 32
    NCHUNKS = EPT // CHUNK
    ZROWS = 40
    AROWS = N2 + ZROWS
    NZCHUNKS = AROWS // ZROWS
    NOCHUNKS = N2 // ZROWS
    assert NC == 2 and EPT * NS == E and CHUNK * NCHUNKS == EPT
    assert CHUNK % 16 == 0 and ZROWS % 8 == 0 and NCHUNKS % 2 == 1
    assert NZCHUNKS * ZROWS == AROWS and NOCHUNKS * ZROWS == N2

    mesh = plsc.VectorSubcoreMesh(core_axis_name="c", subcore_axis_name="s")

    @functools.partial(
        pl.kernel,
        out_type=jax.ShapeDtypeStruct((N, H), jnp.float32),
        mesh=mesh,
        scratch_types=[
            pltpu.VMEM((EPT,), jnp.int32),
            pltpu.VMEM((EPT,), jnp.int32),
            [pltpu.VMEM((CHUNK,), jnp.int32) for _ in range(2)],
            [pltpu.VMEM((CHUNK,), jnp.int32) for _ in range(2)],
            pltpu.VMEM((4, CHUNK), jnp.int32),
            [pltpu.VMEM((CHUNK, H), jnp.float32) for _ in range(2)],
            [pltpu.VMEM((CHUNK, H), jnp.float32) for _ in range(2)],
            [pltpu.VMEM((CHUNK, H), jnp.float32) for _ in range(2)],
            [pltpu.VMEM((CHUNK, H), jnp.float32) for _ in range(2)],
            pltpu.VMEM((ZROWS, H), jnp.float32),
            pltpu.VMEM_SHARED((AROWS, H), jnp.float32),
            [pltpu.SemaphoreType.DMA for _ in range(2)],
            [pltpu.SemaphoreType.DMA for _ in range(2)],
        ],
    )
    def edge_kernel(a_hbm, b_hbm, c_hbm, dst_hbm, src_hbm, out_hbm,
                    dib_v, sib_v, di, si, dl4_v, ar, br, cr, mg,
                    zb_v, m_sh, sg, ss):
        cid = lax.axis_index("c")
        sid = lax.axis_index("s")
        nvec = H // 16

        zval = jnp.zeros((16,), jnp.float32)

        def zb_body(j, _):
            zb_v[j // nvec, pl.ds((j % nvec) * 16, 16)] = zval
            return 0

        lax.fori_loop(0, ZROWS * nvec, zb_body, 0)
        for k in range((NZCHUNKS + NS - 1) // NS):
            zc = sid + k * NS

            @pl.when(zc < NZCHUNKS)
            def _():
                pltpu.sync_copy(
                    zb_v, m_sh.at[pl.ds(pl.multiple_of(zc * ZROWS, 8), ZROWS)])
        plsc.subcore_barrier()

        base_t = sid * EPT
        nbase = cid * N2
        pltpu.sync_copy(dst_hbm.at[pl.ds(base_t, EPT)], dib_v)
        pltpu.sync_copy(src_hbm.at[pl.ds(base_t, EPT)], sib_v)

        def issue(i, b):
            ioff = i * CHUNK
            islot = jnp.bitwise_and(i, 3)
            for k in range(CHUNK // 16):
                so = pl.ds(k * 16, 16)
                sl = pl.ds(ioff + k * 16, 16)
                dv = dib_v[sl]
                di[b][so] = dv
                si[b][so] = sib_v[sl]
                loc = dv - nbase
                ok = jnp.logical_and(loc >= 0, loc < N2)
                dl4_v[islot, so] = jnp.where(ok, loc, N2)
            pltpu.async_copy(a_hbm.at[di[b]], ar[b], sg[b])
            pltpu.async_copy(b_hbm.at[si[b]], br[b], sg[b])
            pltpu.async_copy(
                c_hbm.at[pl.ds(pl.multiple_of(base_t + ioff, 8), CHUNK)],
                cr[b], sg[b])

        def wait_gathers(b):
            pltpu.make_async_copy(a_hbm.at[di[b]], ar[b], sg[b]).wait()
            pltpu.make_async_copy(b_hbm.at[si[b]], br[b], sg[b]).wait()
            pltpu.make_async_copy(
                c_hbm.at[pl.ds(0, CHUNK)], cr[b], sg[b]).wait()

        def wait_scatter(b):
            pltpu.make_async_copy(mg[b], m_sh.at[dl4_v.at[0]], ss[b]).wait()

        def compute(b):
            def row_body(r, _):
                for v in range(nvec):
                    sl = pl.ds(v * 16, 16)
                    z = ar[b][r, sl] + br[b][r, sl] + cr[b][r, sl]
                    e = jnp.exp(jnp.minimum(z * -2.0, 80.0))
                    mg[b][r, sl] = (1.0 - e) / (1.0 + e)
                return 0

            lax.fori_loop(0, CHUNK, row_body, 0)

        def scatter(i, b):
            pltpu.async_copy(
                mg[b], m_sh.at[dl4_v.at[jnp.bitwise_and(i, 3)]], ss[b],
                add=True)

        issue(0, 0)

        def grp_body(g, _):
            for b in range(2):
                i = g * 2 + b

                @pl.when(i + 1 < NCHUNKS)
                def _():
                    issue(i + 1, 1 - b)

                wait_gathers(b)

                @pl.when(i >= 2)
                def _():
                    wait_scatter(b)

                compute(b)
                scatter(i, b)
            return 0

        lax.fori_loop(0, NCHUNKS // 2, grp_body, 0)
        wait_gathers(0)
        wait_scatter(0)
        compute(0)
        scatter(NCHUNKS - 1, 0)
        wait_scatter(0)
        wait_scatter(1)
        plsc.subcore_barrier()

        for k in range((NOCHUNKS + NS - 1) // NS):
            zc = sid + k * NS

            @pl.when(zc < NOCHUNKS)
            def _():
                off = pl.multiple_of(zc * ZROWS, 8)
                pltpu.sync_copy(m_sh.at[pl.ds(off, ZROWS)], zb_v)
                pltpu.sync_copy(
                    zb_v,
                    out_hbm.at[pl.ds(pl.multiple_of(nbase + off, 8), ZROWS)])

    return edge_kernel


def kernel(x, edge_index, edge_attr, W_msg, b_msg, W_upd, b_upd, W_out, b_out):
    N, H = x.shape
    E = edge_index.shape[1]
    ED = edge_attr.shape[1]

    W1 = W_msg[:H]
    W2 = W_msg[H:2 * H]
    W3 = W_msg[2 * H:]
    Wum = W_upd[:H]
    Wuh = W_upd[H:]
    dst = edge_index[0]
    src = edge_index[1]

    C = _edge_bias_table(edge_attr, W3, b_msg, E, ED, H)
    A0, B0 = _proj_ab(x, W1, W2, N, H)

    edge_kernel = _make_edge_kernel(N, E, H)

    def body(carry, _):
        m, h, A, B = carry
        P = edge_kernel(A, B, C, dst, src)
        m, h, A, B = _update_mid(P, m, h, Wum, Wuh, b_upd, W1, W2, N, H)
        return (m, h, A, B), None

    m0 = jnp.zeros((N, H), jnp.float32)
    (m, h, A, B), _ = lax.scan(body, (m0, x, A0, B0), None, length=2)
    out = _readout(h, W_out, b_out, N, H)
    return out.reshape(H)


# speedup vs baseline: 3.7829x; 1.0244x over previous
import functools

import jax
import jax.numpy as jnp
from jax import lax
from jax.experimental import pallas as pl
from jax.experimental.pallas import tpu as pltpu
from jax.experimental.pallas import tpu_sc as plsc


def _c_body(ea_ref, w_ref, b_ref, out_ref):
    out_ref[...] = (
        jnp.dot(ea_ref[...], w_ref[...], preferred_element_type=jnp.float32)
        + b_ref[...]
    )


def _edge_bias_table(edge_attr, W3, b_msg, E, ED, H):
    EB = 8000
    return pl.pallas_call(
        _c_body,
        grid=(E // EB,),
        in_specs=[
            pl.BlockSpec((EB, ED), lambda i: (i, 0)),
            pl.BlockSpec((ED, H), lambda i: (0, 0)),
            pl.BlockSpec((1, H), lambda i: (0, 0)),
        ],
        out_specs=pl.BlockSpec((EB, H), lambda i: (i, 0)),
        out_shape=jax.ShapeDtypeStruct((E, H), jnp.float32),
    )(edge_attr, W3, b_msg.reshape(1, H))


def _ab_body(h_ref, w1_ref, w2_ref, a_ref, b_ref):
    h = h_ref[...]
    a_ref[...] = jnp.dot(h, w1_ref[...], preferred_element_type=jnp.float32)
    b_ref[...] = jnp.dot(h, w2_ref[...], preferred_element_type=jnp.float32)


def _proj_ab(h, W1, W2, N, H):
    NB = 2000
    nspec = pl.BlockSpec((NB, H), lambda i: (i, 0))
    nshape = jax.ShapeDtypeStruct((N, H), jnp.float32)
    return pl.pallas_call(
        _ab_body,
        grid=(N // NB,),
        in_specs=[
            nspec,
            pl.BlockSpec((H, H), lambda i: (0, 0)),
            pl.BlockSpec((H, H), lambda i: (0, 0)),
        ],
        out_specs=[nspec, nspec],
        out_shape=[nshape, nshape],
    )(h, W1, W2)


def _upd_mid_body(p_ref, mp_ref, h_ref, wum_ref, wuh_ref, bu_ref, w1_ref,
                  w2_ref, m_ref, hn_ref, a_ref, b_ref):
    m = mp_ref[...] + p_ref[...]
    hn = jnp.tanh(
        jnp.dot(m, wum_ref[...], preferred_element_type=jnp.float32)
        + jnp.dot(h_ref[...], wuh_ref[...], preferred_element_type=jnp.float32)
        + bu_ref[...]
    )
    m_ref[...] = m
    hn_ref[...] = hn
    a_ref[...] = jnp.dot(hn, w1_ref[...], preferred_element_type=jnp.float32)
    b_ref[...] = jnp.dot(hn, w2_ref[...], preferred_element_type=jnp.float32)


def _update_mid(P, m_prev, h, Wum, Wuh, b_upd, W1, W2, N, H):
    NB = 2000
    wspec = pl.BlockSpec((H, H), lambda i: (0, 0))
    nspec = pl.BlockSpec((NB, H), lambda i: (i, 0))
    nshape = jax.ShapeDtypeStruct((N, H), jnp.float32)
    return pl.pallas_call(
        _upd_mid_body,
        grid=(N // NB,),
        in_specs=[
            nspec, nspec, nspec, wspec, wspec,
            pl.BlockSpec((1, H), lambda i: (0, 0)),
            wspec, wspec,
        ],
        out_specs=[nspec, nspec, nspec, nspec],
        out_shape=[nshape, nshape, nshape, nshape],
    )(P, m_prev, h, Wum, Wuh, b_upd.reshape(1, H), W1, W2)


def _readout_body(h_ref, wo_ref, bo_ref, out_ref, acc_ref):
    i = pl.program_id(0)
    part = jnp.sum(h_ref[...], axis=0, keepdims=True)

    @pl.when(i == 0)
    def _():
        acc_ref[...] = part

    @pl.when(i > 0)
    def _():
        acc_ref[...] = acc_ref[...] + part

    @pl.when(i == pl.num_programs(0) - 1)
    def _():
        out_ref[...] = jnp.tanh(
            jnp.dot(acc_ref[...], wo_ref[...], preferred_element_type=jnp.float32)
            + bo_ref[...]
        )


def _readout(h, W_out, b_out, N, H):
    NB = 2000
    return pl.pallas_call(
        _readout_body,
        grid=(N // NB,),
        in_specs=[
            pl.BlockSpec((NB, H), lambda i: (i, 0)),
            pl.BlockSpec((H, H), lambda i: (0, 0)),
            pl.BlockSpec((1, H), lambda i: (0, 0)),
        ],
        out_specs=pl.BlockSpec((1, H), lambda i: (0, 0)),
        out_shape=jax.ShapeDtypeStruct((1, H), jnp.float32),
        scratch_shapes=[pltpu.VMEM((1, H), jnp.float32)],
    )(h, W_out, b_out.reshape(1, H))


def _make_edge_kernel(N, E, H):
    info = plsc.get_sparse_core_info()
    NC, NS = info.num_cores, info.num_subcores
    N2 = N // 2
    EPT = E // NS
    CHUNK = 32
    NCHUNKS = EPT // CHUNK
    ZROWS = 40
    AROWS = N2 + ZROWS
    NZCHUNKS = AROWS // ZROWS
    NOCHUNKS = N2 // ZROWS
    assert NC == 2 and EPT * NS == E and CHUNK * NCHUNKS == EPT
    assert CHUNK % 16 == 0 and ZROWS % 8 == 0 and NCHUNKS % 2 == 1
    assert NZCHUNKS * ZROWS == AROWS and NOCHUNKS * ZROWS == N2

    mesh = plsc.VectorSubcoreMesh(core_axis_name="c", subcore_axis_name="s")

    @functools.partial(
        pl.kernel,
        out_type=jax.ShapeDtypeStruct((N, H), jnp.float32),
        mesh=mesh,
        scratch_types=[
            pltpu.VMEM((EPT,), jnp.int32),
            pltpu.VMEM((EPT,), jnp.int32),
            [pltpu.VMEM((CHUNK,), jnp.int32) for _ in range(2)],
            [pltpu.VMEM((CHUNK,), jnp.int32) for _ in range(2)],
            pltpu.VMEM((4, CHUNK), jnp.int32),
            [pltpu.VMEM((CHUNK, H), jnp.float32) for _ in range(2)],
            [pltpu.VMEM((CHUNK, H), jnp.float32) for _ in range(2)],
            [pltpu.VMEM((CHUNK, H), jnp.float32) for _ in range(2)],
            [pltpu.VMEM((CHUNK, H), jnp.float32) for _ in range(2)],
            pltpu.VMEM((ZROWS, H), jnp.float32),
            pltpu.VMEM_SHARED((AROWS, H), jnp.float32),
            [pltpu.SemaphoreType.DMA for _ in range(2)],
            [pltpu.SemaphoreType.DMA for _ in range(2)],
        ],
    )
    def edge_kernel(a_hbm, b_hbm, c_hbm, dst_hbm, src_hbm, out_hbm,
                    dib_v, sib_v, di, si, dl4_v, ar, br, cr, mg,
                    zb_v, m_sh, sg, ss):
        cid = lax.axis_index("c")
        sid = lax.axis_index("s")
        nvec = H // 16

        zval = jnp.zeros((16,), jnp.float32)

        def zb_body(j, _):
            zb_v[j // nvec, pl.ds((j % nvec) * 16, 16)] = zval
            return 0

        lax.fori_loop(0, ZROWS * nvec, zb_body, 0)
        for k in range((NZCHUNKS + NS - 1) // NS):
            zc = sid + k * NS

            @pl.when(zc < NZCHUNKS)
            def _():
                pltpu.sync_copy(
                    zb_v, m_sh.at[pl.ds(pl.multiple_of(zc * ZROWS, 8), ZROWS)])
        plsc.subcore_barrier()

        base_t = sid * EPT
        nbase = cid * N2
        pltpu.sync_copy(dst_hbm.at[pl.ds(base_t, EPT)], dib_v)
        pltpu.sync_copy(src_hbm.at[pl.ds(base_t, EPT)], sib_v)

        def issue(i, b):
            ioff = i * CHUNK
            islot = jnp.bitwise_and(i, 3)
            for k in range(CHUNK // 16):
                so = pl.ds(k * 16, 16)
                sl = pl.ds(ioff + k * 16, 16)
                dv = dib_v[sl]
                di[b][so] = dv
                si[b][so] = sib_v[sl]
                loc = dv - nbase
                ok = jnp.logical_and(loc >= 0, loc < N2)
                dl4_v[islot, so] = jnp.where(ok, loc, N2)
            pltpu.async_copy(a_hbm.at[di[b]], ar[b], sg[b])
            pltpu.async_copy(b_hbm.at[si[b]], br[b], sg[b])
            pltpu.async_copy(
                c_hbm.at[pl.ds(pl.multiple_of(base_t + ioff, 8), CHUNK)],
                cr[b], sg[b])

        def wait_gathers(b):
            pltpu.make_async_copy(a_hbm.at[di[b]], ar[b], sg[b]).wait()
            pltpu.make_async_copy(b_hbm.at[si[b]], br[b], sg[b]).wait()
            pltpu.make_async_copy(
                c_hbm.at[pl.ds(0, CHUNK)], cr[b], sg[b]).wait()

        def wait_scatter(b):
            pltpu.make_async_copy(mg[b], m_sh.at[dl4_v.at[0]], ss[b]).wait()

        def compute(b):
            def row_body(r2, _):
                for u in range(2):
                    r = r2 * 2 + u
                    for v in range(nvec):
                        sl = pl.ds(v * 16, 16)
                        zn = ar[b][r, sl] + br[b][r, sl] + cr[b][r, sl]
                        e = jnp.exp(jnp.minimum(zn, 80.0))
                        mg[b][r, sl] = (1.0 - e) / (1.0 + e)
                return 0

            lax.fori_loop(0, CHUNK // 2, row_body, 0)

        def scatter(i, b):
            pltpu.async_copy(
                mg[b], m_sh.at[dl4_v.at[jnp.bitwise_and(i, 3)]], ss[b],
                add=True)

        issue(0, 0)

        def grp_body(g, _):
            for b in range(2):
                i = g * 2 + b

                @pl.when(i + 1 < NCHUNKS)
                def _():
                    issue(i + 1, 1 - b)

                wait_gathers(b)

                @pl.when(i >= 2)
                def _():
                    wait_scatter(b)

                compute(b)
                scatter(i, b)
            return 0

        lax.fori_loop(0, NCHUNKS // 2, grp_body, 0)
        wait_gathers(0)
        wait_scatter(0)
        compute(0)
        scatter(NCHUNKS - 1, 0)
        wait_scatter(0)
        wait_scatter(1)
        plsc.subcore_barrier()

        for k in range((NOCHUNKS + NS - 1) // NS):
            zc = sid + k * NS

            @pl.when(zc < NOCHUNKS)
            def _():
                off = pl.multiple_of(zc * ZROWS, 8)
                pltpu.sync_copy(m_sh.at[pl.ds(off, ZROWS)], zb_v)
                pltpu.sync_copy(
                    zb_v,
                    out_hbm.at[pl.ds(pl.multiple_of(nbase + off, 8), ZROWS)])

    return edge_kernel


def kernel(x, edge_index, edge_attr, W_msg, b_msg, W_upd, b_upd, W_out, b_out):
    N, H = x.shape
    E = edge_index.shape[1]
    ED = edge_attr.shape[1]

    W1 = W_msg[:H] * -2.0
    W2 = W_msg[H:2 * H] * -2.0
    W3 = W_msg[2 * H:] * -2.0
    Wum = W_upd[:H]
    Wuh = W_upd[H:]
    dst = edge_index[0]
    src = edge_index[1]

    C = _edge_bias_table(edge_attr, W3, b_msg * -2.0, E, ED, H)
    A0, B0 = _proj_ab(x, W1, W2, N, H)

    edge_kernel = _make_edge_kernel(N, E, H)

    def body(carry, _):
        m, h, A, B = carry
        P = edge_kernel(A, B, C, dst, src)
        m, h, A, B = _update_mid(P, m, h, Wum, Wuh, b_upd, W1, W2, N, H)
        return (m, h, A, B), None

    m0 = jnp.zeros((N, H), jnp.float32)
    (m, h, A, B), _ = lax.scan(body, (m0, x, A0, B0), None, length=2)
    out = _readout(h, W_out, b_out, N, H)
    return out.reshape(H)
